# baseline (speedup 1.0000x reference)
import functools

import jax
import jax.numpy as jnp
from jax import lax
from jax.experimental import pallas as pl
from jax.experimental.pallas import tpu as pltpu
from jax.experimental.pallas import tpu_sc as plsc

N_NODES = 10000
N_EDGES = 160000
D = 256

NC = 2
NS = 16
NW = NC * NS

CHUNK = 80
NBUF = 4
E_PAD = 163840
CH_PER_W = E_PAD // (NW * CHUNK)
ROWS_PER_W = E_PAD // NW


def _mm_table_kernel(h_ref, w1_ref, b_ref, o_ref):
    o_ref[...] = lax.dot_general(
        h_ref[...], w1_ref[...],
        (((1,), (1,)), ((), ())),
        preferred_element_type=jnp.float32,
    ) + b_ref[...]


def _mm_edges_kernel(e_ref, g_ref, w2_ref, o_ref):
    o_ref[...] = lax.dot_general(
        e_ref[...], w2_ref[...],
        (((1,), (1,)), ((), ())),
        preferred_element_type=jnp.float32,
    ) + g_ref[...]


@functools.cache
def _make_sc_gather():
    @functools.partial(
        pl.kernel,
        out_type=jax.ShapeDtypeStruct((E_PAD, D), jnp.float32),
        mesh=plsc.VectorSubcoreMesh(
            core_axis_name="c", subcore_axis_name="s",
            num_cores=NC, num_subcores=NS,
        ),
        scratch_types=(
            [pltpu.VMEM((CH_PER_W, CHUNK), jnp.int32)]
            + [pltpu.VMEM((CHUNK, D), jnp.float32)] * NBUF
            + [pltpu.SemaphoreType.DMA] * (2 * NBUF)
        ),
    )
    def _sc_gather(table_hbm, idx_hbm, out_hbm, idx_v, *scr):
        bufs = scr[:NBUF]
        gsem = scr[NBUF:2 * NBUF]
        ssem = scr[2 * NBUF:]
        wid = lax.axis_index("s") * NC + lax.axis_index("c")
        base = wid * ROWS_PER_W
        pltpu.sync_copy(idx_hbm.at[wid], idx_v)

        def g_copy(j, k):
            return pltpu.make_async_copy(table_hbm.at[idx_v.at[j]], bufs[k],
                                         gsem[k])

        def s_copy(j, k):
            return pltpu.make_async_copy(
                bufs[k], out_hbm.at[pl.ds(base + j * CHUNK, CHUNK)], ssem[k])

        def step(j, k, wait_prev_store, gather_ahead):
            g_copy(j, k).wait()
            s_copy(j, k).start()
            if wait_prev_store:
                s_copy(j - 2, (k + 2) % NBUF).wait()
            if gather_ahead:
                g_copy(j + 2, (k + 2) % NBUF).start()

        g_copy(0, 0).start()
        g_copy(1, 1).start()
        step(0, 0, False, True)
        step(1, 1, False, True)
        step(2, 2, True, True)
        step(3, 3, True, True)

        def body(i, carry):
            j = NBUF * i
            step(j + 0, 0, True, True)
            step(j + 1, 1, True, True)
            step(j + 2, 2, True, True)
            step(j + 3, 3, True, True)
            return carry

        lax.fori_loop(1, CH_PER_W // NBUF - 1, body, 0)

        j = CH_PER_W - NBUF
        step(j + 0, 0, True, True)
        step(j + 1, 1, True, True)
        step(j + 2, 2, True, False)
        step(j + 3, 3, True, False)
        s_copy(CH_PER_W - 2, 2).wait()
        s_copy(CH_PER_W - 1, 3).wait()

    return _sc_gather


def kernel(H, E, r_embed, heads, queries, W, b):
    w1 = W[:, :D]
    w2 = W[:, D:]
    b2 = b.reshape(1, D)

    table = pl.pallas_call(
        _mm_table_kernel,
        out_shape=jax.ShapeDtypeStruct((N_NODES, D), jnp.float32),
    )(H, w1, b2)

    heads_pad = jnp.concatenate(
        [heads, jnp.zeros((E_PAD - N_EDGES,), jnp.int32)]
    ).reshape(NW, CH_PER_W, CHUNK)
    gathered = _make_sc_gather()(table, heads_pad)

    blk = 2000
    grid = (N_EDGES // blk,)
    out = pl.pallas_call(
        _mm_edges_kernel,
        grid=grid,
        in_specs=[
            pl.BlockSpec((blk, D), lambda i: (i, 0)),
            pl.BlockSpec((blk, D), lambda i: (i, 0)),
            pl.BlockSpec((D, D), lambda i: (0, 0)),
        ],
        out_specs=pl.BlockSpec((blk, D), lambda i: (i, 0)),
        out_shape=jax.ShapeDtypeStruct((N_EDGES, D), jnp.float32),
    )(E, gathered, w2)
    return out


# speedup vs baseline: 1.6711x; 1.6711x over previous
import functools

import jax
import jax.numpy as jnp
from jax import lax
from jax.experimental import pallas as pl
from jax.experimental.pallas import tpu as pltpu
from jax.experimental.pallas import tpu_sc as plsc

N_NODES = 10000
N_EDGES = 160000
D = 256

NC = 2
NS = 16
NW = NC * NS

CHUNK = 80
NBUF = 4
E_PAD = 163840
CH_PER_W = E_PAD // (NW * CHUNK)
ROWS_PER_W = E_PAD // NW


def _mm_table_kernel(h_ref, w1_ref, b_ref, o_ref):
    o_ref[...] = lax.dot_general(
        h_ref[...], w1_ref[...],
        (((1,), (1,)), ((), ())),
        preferred_element_type=jnp.float32,
    ) + b_ref[...]


def _mm_edges_kernel(e_ref, g_ref, w2_ref, o_ref):
    o_ref[...] = lax.dot_general(
        e_ref[...], w2_ref[...],
        (((1,), (1,)), ((), ())),
        preferred_element_type=jnp.float32,
    ) + g_ref[...]


@functools.cache
def _make_sc_gather():
    @functools.partial(
        pl.kernel,
        out_type=jax.ShapeDtypeStruct((E_PAD, D), jnp.float32),
        mesh=plsc.VectorSubcoreMesh(
            core_axis_name="c", subcore_axis_name="s",
            num_cores=NC, num_subcores=NS,
        ),
        scratch_types=(
            [pltpu.VMEM((CH_PER_W, CHUNK), jnp.int32)]
            + [pltpu.VMEM((CHUNK, D), jnp.float32)] * NBUF
            + [pltpu.SemaphoreType.DMA] * (2 * NBUF)
        ),
    )
    def _sc_gather(table_hbm, idx_hbm, out_hbm, idx_v, *scr):
        bufs = scr[:NBUF]
        gsem = scr[NBUF:2 * NBUF]
        ssem = scr[2 * NBUF:]
        wid = lax.axis_index("s") * NC + lax.axis_index("c")
        base = wid * ROWS_PER_W
        pltpu.sync_copy(idx_hbm.at[wid], idx_v)

        def g_copy(j, k):
            return pltpu.make_async_copy(table_hbm.at[idx_v.at[j]], bufs[k],
                                         gsem[k])

        def s_copy(j, k):
            return pltpu.make_async_copy(
                bufs[k], out_hbm.at[pl.ds(base + j * CHUNK, CHUNK)], ssem[k])

        def step(j, k, wait_prev_store, gather_ahead):
            g_copy(j, k).wait()
            s_copy(j, k).start()
            if wait_prev_store:
                s_copy(j - 2, (k + 2) % NBUF).wait()
            if gather_ahead:
                g_copy(j + 2, (k + 2) % NBUF).start()

        g_copy(0, 0).start()
        g_copy(1, 1).start()
        step(0, 0, False, True)
        step(1, 1, False, True)
        step(2, 2, True, True)
        step(3, 3, True, True)

        def body(i, carry):
            j = NBUF * i
            step(j + 0, 0, True, True)
            step(j + 1, 1, True, True)
            step(j + 2, 2, True, True)
            step(j + 3, 3, True, True)
            return carry

        lax.fori_loop(1, CH_PER_W // NBUF - 1, body, 0)

        j = CH_PER_W - NBUF
        step(j + 0, 0, True, True)
        step(j + 1, 1, True, True)
        step(j + 2, 2, True, False)
        step(j + 3, 3, True, False)
        s_copy(CH_PER_W - 2, 2).wait()
        s_copy(CH_PER_W - 1, 3).wait()

    return _sc_gather


def kernel(H, E, r_embed, heads, queries, W, b):
    w1 = W[:, :D]
    w2 = W[:, D:]
    b2 = b.reshape(1, D)

    table = pl.pallas_call(
        _mm_table_kernel,
        out_shape=jax.ShapeDtypeStruct((N_NODES, D), jnp.float32),
    )(H, w1, b2)

    pad_idx = (jnp.arange(E_PAD - N_EDGES, dtype=jnp.int32) * 16) % N_NODES
    heads_pad = jnp.concatenate([heads, pad_idx]).reshape(
        NW, CH_PER_W, CHUNK)
    gathered = _make_sc_gather()(table, heads_pad)

    blk = 2000
    grid = (N_EDGES // blk,)
    out = pl.pallas_call(
        _mm_edges_kernel,
        grid=grid,
        in_specs=[
            pl.BlockSpec((blk, D), lambda i: (i, 0)),
            pl.BlockSpec((blk, D), lambda i: (i, 0)),
            pl.BlockSpec((D, D), lambda i: (0, 0)),
        ],
        out_specs=pl.BlockSpec((blk, D), lambda i: (i, 0)),
        out_shape=jax.ShapeDtypeStruct((N_EDGES, D), jnp.float32),
    )(E, gathered, w2)
    return out
